# baseline (speedup 1.0000x reference)
import functools

import jax
import jax.numpy as jnp
from jax import lax
from jax.experimental import pallas as pl
from jax.experimental.pallas import tpu as pltpu
from jax.experimental.pallas import tpu_sc as plsc

NUM_CORES = 2
NUM_SUBCORES = 16
NUM_WORKERS = NUM_CORES * NUM_SUBCORES


def _make_kernel(bsz, seq, dim):
    assert bsz % NUM_WORKERS == 0
    bw = bsz // NUM_WORKERS
    assert seq % 2 == 0

    mesh = plsc.VectorSubcoreMesh(core_axis_name="c", subcore_axis_name="s")

    @functools.partial(
        pl.kernel,
        out_type=jax.ShapeDtypeStruct((bsz, seq, dim), jnp.float32),
        mesh=mesh,
        compiler_params=pltpu.CompilerParams(use_tc_tiling_on_sc=False),
        scratch_types=[
            pltpu.VMEM((seq, bw), jnp.int32),
            pltpu.VMEM((bw, dim), jnp.float32),
            pltpu.VMEM((bw, dim), jnp.float32),
            pltpu.SemaphoreType.DMA,
            pltpu.SemaphoreType.DMA,
            pltpu.SemaphoreType.DMA,
            pltpu.SemaphoreType.DMA,
        ],
    )
    def gather_kernel(idx_hbm, table_hbm, out_hbm, idx_v, rows0, rows1,
                      g0, g1, w0, w1):
        wid = lax.axis_index("s") * NUM_CORES + lax.axis_index("c")
        base = wid * bw
        pltpu.sync_copy(idx_hbm.at[:, pl.ds(base, bw)], idx_v)

        def fire_g(t, buf, sem):
            pltpu.async_copy(table_hbm.at[idx_v.at[t]], buf, sem)

        def drain_g(t, buf, sem):
            pltpu.make_async_copy(table_hbm.at[idx_v.at[t]], buf, sem).wait()

        def fire_wb(t, buf, sem):
            pltpu.async_copy(buf, out_hbm.at[pl.ds(base, bw), t], sem)

        def drain_wb(t, buf, sem):
            pltpu.make_async_copy(
                buf, out_hbm.at[pl.ds(base, bw), t], sem
            ).wait()

        fire_g(0, rows0, g0)

        def body(i, carry):
            t0 = 2 * i
            t1 = t0 + 1

            drain_g(t0, rows0, g0)
            fire_wb(t0, rows0, w0)

            @pl.when(i > 0)
            def _():
                drain_wb(t0 - 1, rows1, w1)

            fire_g(t1, rows1, g1)

            drain_g(t1, rows1, g1)
            fire_wb(t1, rows1, w1)

            @pl.when(i < seq // 2 - 1)
            def _():
                drain_wb(t0, rows0, w0)
                fire_g(t1 + 1, rows0, g0)

            return carry

        lax.fori_loop(0, seq // 2, body, 0)

        drain_wb(seq - 2, rows0, w0)
        drain_wb(seq - 1, rows1, w1)

    return gather_kernel


def kernel(token_ids, weight):
    b, t = token_ids.shape
    dim = weight.shape[1]
    tids_t = token_ids.astype(jnp.int32).T
    return _make_kernel(b, t, dim)(tids_t, weight)


# speedup vs baseline: 1.0139x; 1.0139x over previous
import functools

import jax
import jax.numpy as jnp
from jax import lax
from jax.experimental import pallas as pl
from jax.experimental.pallas import tpu as pltpu
from jax.experimental.pallas import tpu_sc as plsc

NUM_CORES = 2
NUM_SUBCORES = 16
NUM_WORKERS = NUM_CORES * NUM_SUBCORES
L = 16


def _make_kernel(bsz, seq, dim):
    assert bsz % NUM_WORKERS == 0
    bw = bsz // NUM_WORKERS
    assert seq % 2 == 0 and bw % L == 0 and dim % L == 0
    groups = bw // L
    qs = dim // L

    mesh = plsc.VectorSubcoreMesh(core_axis_name="c", subcore_axis_name="s")

    @functools.partial(
        pl.kernel,
        out_type=jax.ShapeDtypeStruct((bsz, seq, dim), jnp.float32),
        mesh=mesh,
        scratch_types=[
            pltpu.VMEM((seq, bw), jnp.int32),
            pltpu.VMEM((bw,), jnp.int32),
            pltpu.VMEM((bw,), jnp.int32),
            pltpu.VMEM((bw, 2 * dim), jnp.float32),
            pltpu.VMEM((bw, 2 * dim), jnp.float32),
            pltpu.VMEM((bw, dim), jnp.float32),
            pltpu.VMEM((bw, dim), jnp.float32),
            pltpu.SemaphoreType.DMA,
            pltpu.SemaphoreType.DMA,
            pltpu.SemaphoreType.DMA,
            pltpu.SemaphoreType.DMA,
        ],
    )
    def gather_kernel(idx_hbm, table_hbm, out_hbm, idx_v, p0, p1,
                      rows0, rows1, outb0, outb1, g0, g1, w0, w1):
        wid = lax.axis_index("s") * NUM_CORES + lax.axis_index("c")
        base = wid * bw
        pltpu.sync_copy(idx_hbm.at[:, pl.ds(base, bw)], idx_v)

        def prep(t, p_v):
            for g in range(groups):
                p_v[pl.ds(g * L, L)] = idx_v[t, pl.ds(g * L, L)] >> 1

        def fire_g(p_v, buf, sem):
            pltpu.async_copy(table_hbm.at[p_v], buf, sem)

        def drain_g(p_v, buf, sem):
            pltpu.make_async_copy(table_hbm.at[p_v], buf, sem).wait()

        def select(t, buf, outb):
            for g in range(groups):
                hvec = idx_v[t, pl.ds(g * L, L)] & 1
                for j in range(L):
                    r = g * L + j
                    hj = hvec.at[jnp.full((L,), j, jnp.int32)].get(
                        mode="promise_in_bounds")
                    hf = hj.astype(jnp.float32)
                    for q in range(qs):
                        lo = buf[r, pl.ds(q * L, L)]
                        hi = buf[r, pl.ds(dim + q * L, L)]
                        outb[r, pl.ds(q * L, L)] = lo + (hi - lo) * hf

        def fire_wb(t, outb, sem):
            pltpu.async_copy(outb, out_hbm.at[pl.ds(base, bw), t], sem)

        def drain_wb(t, outb, sem):
            pltpu.make_async_copy(
                outb, out_hbm.at[pl.ds(base, bw), t], sem
            ).wait()

        prep(0, p0)
        fire_g(p0, rows0, g0)

        def body(i, carry):
            t0 = 2 * i
            t1 = t0 + 1

            prep(t1, p1)
            fire_g(p1, rows1, g1)

            drain_g(p0, rows0, g0)

            @pl.when(i > 0)
            def _():
                drain_wb(t0 - 2, outb0, w0)

            select(t0, rows0, outb0)
            fire_wb(t0, outb0, w0)

            @pl.when(i < seq // 2 - 1)
            def _():
                prep(t0 + 2, p0)
                fire_g(p0, rows0, g0)

            drain_g(p1, rows1, g1)

            @pl.when(i > 0)
            def _():
                drain_wb(t1 - 2, outb1, w1)

            select(t1, rows1, outb1)
            fire_wb(t1, outb1, w1)

            return carry

        lax.fori_loop(0, seq // 2, body, 0)

        drain_wb(seq - 2, outb0, w0)
        drain_wb(seq - 1, outb1, w1)

    return gather_kernel


def kernel(token_ids, weight):
    b, t = token_ids.shape
    n, dim = weight.shape
    tids_t = token_ids.astype(jnp.int32).T
    wpair = weight.reshape(n // 2, 2 * dim)
    return _make_kernel(b, t, dim)(tids_t, wpair)


# speedup vs baseline: 1.0309x; 1.0168x over previous
import functools

import jax
import jax.numpy as jnp
from jax import lax
from jax.experimental import pallas as pl
from jax.experimental.pallas import tpu as pltpu
from jax.experimental.pallas import tpu_sc as plsc

NUM_CORES = 2
NUM_SUBCORES = 16
NUM_WORKERS = NUM_CORES * NUM_SUBCORES

GATHER_ROWS = 128
GATHERS_PER_ROUND = 5
ROUND_ROWS = GATHER_ROWS * GATHERS_PER_ROUND


def _make_kernel(total, dim):
    assert total % (NUM_WORKERS * ROUND_ROWS) == 0
    per_worker = total // NUM_WORKERS
    gathers = per_worker // GATHER_ROWS
    rounds = per_worker // ROUND_ROWS
    assert rounds % 2 == 0 and rounds >= 4

    mesh = plsc.VectorSubcoreMesh(core_axis_name="c", subcore_axis_name="s")

    @functools.partial(
        pl.kernel,
        out_type=jax.ShapeDtypeStruct((total, dim), jnp.float32),
        mesh=mesh,
        compiler_params=pltpu.CompilerParams(use_tc_tiling_on_sc=False),
        scratch_types=[
            pltpu.VMEM((gathers, GATHER_ROWS), jnp.int32),
            pltpu.VMEM((ROUND_ROWS, dim), jnp.float32),
            pltpu.VMEM((ROUND_ROWS, dim), jnp.float32),
            pltpu.SemaphoreType.DMA,
            pltpu.SemaphoreType.DMA,
            pltpu.SemaphoreType.DMA,
            pltpu.SemaphoreType.DMA,
        ],
    )
    def gather_kernel(idx_hbm, table_hbm, out_hbm, idx_v, rows0, rows1,
                      g0, g1, w0, w1):
        wid = lax.axis_index("s") * NUM_CORES + lax.axis_index("c")
        pltpu.sync_copy(idx_hbm.at[wid], idx_v)
        row_base = wid * per_worker

        def fire_round(j, buf, sem):
            for g in range(GATHERS_PER_ROUND):
                pltpu.async_copy(
                    table_hbm.at[idx_v.at[j * GATHERS_PER_ROUND + g]],
                    buf.at[pl.ds(g * GATHER_ROWS, GATHER_ROWS)],
                    sem,
                )

        def drain_round(j, buf, sem):
            for g in range(GATHERS_PER_ROUND):
                pltpu.make_async_copy(
                    table_hbm.at[idx_v.at[j * GATHERS_PER_ROUND + g]],
                    buf.at[pl.ds(g * GATHER_ROWS, GATHER_ROWS)],
                    sem,
                ).wait()

        def fire_wb(j, buf, sem):
            pltpu.async_copy(
                buf, out_hbm.at[pl.ds(row_base + j * ROUND_ROWS, ROUND_ROWS)], sem
            )

        def drain_wb(j, buf, sem):
            pltpu.make_async_copy(
                buf, out_hbm.at[pl.ds(row_base + j * ROUND_ROWS, ROUND_ROWS)], sem
            ).wait()

        fire_round(0, rows0, g0)

        def body(i, carry):
            r0 = 2 * i
            r1 = r0 + 1

            drain_round(r0, rows0, g0)
            fire_wb(r0, rows0, w0)
            @pl.when(i > 0)
            def _():
                drain_wb(r0 - 1, rows1, w1)

            fire_round(r1, rows1, g1)

            drain_round(r1, rows1, g1)
            fire_wb(r1, rows1, w1)
            @pl.when(i < rounds // 2 - 1)
            def _():
                drain_wb(r0, rows0, w0)
                fire_round(r1 + 1, rows0, g0)

            return carry

        lax.fori_loop(0, rounds // 2, body, 0)

        drain_wb(rounds - 2, rows0, w0)
        drain_wb(rounds - 1, rows1, w1)

    return gather_kernel


def kernel(token_ids, weight):
    b, t = token_ids.shape
    total = b * t
    dim = weight.shape[1]
    per_worker = total // NUM_WORKERS
    idx3d = token_ids.reshape(
        NUM_WORKERS, per_worker // GATHER_ROWS, GATHER_ROWS
    ).astype(jnp.int32)
    out = _make_kernel(total, dim)(idx3d, weight)
    return out.reshape(b, t, dim)
